# baseline (speedup 1.0000x reference)
import functools

import jax
import jax.numpy as jnp
from jax import lax
from jax.experimental import pallas as pl
from jax.experimental.pallas import tpu as pltpu
from jax.experimental.pallas import tpu_sc as plsc

_NC = 2
_NS = 16
_NW = _NC * _NS
_LANES = 16

_B = 4
_SEQ = 8192
_D = 1024
_ROWS_PER_W = _SEQ // _NW
_T = 4
_C = _T * _D
_NTILES = _ROWS_PER_W // _T
_NBUF = 4


def _sc_body(seq_hbm, table_hbm, out_hbm, *scratch):
    tbufs = scratch[0:_NBUF]
    sbufs = [scratch[_NBUF + i * _B:_NBUF + (i + 1) * _B] for i in range(_NBUF)]
    si = scratch[_NBUF + _NBUF * _B:_NBUF + _NBUF * _B + _NBUF]
    so = scratch[_NBUF + _NBUF * _B + _NBUF:]

    wid = lax.axis_index("s") * _NC + lax.axis_index("c")
    base = wid * _ROWS_PER_W * _D

    def in_copies(t, k):
        off = base + t * _C
        yield pltpu.make_async_copy(table_hbm.at[pl.ds(off, _C)], tbufs[k], si[k])
        for b in range(_B):
            soff = b * (_SEQ * _D) + off
            yield pltpu.make_async_copy(seq_hbm.at[pl.ds(soff, _C)], sbufs[k][b], si[k])

    def out_copies(t, k):
        off = base + t * _C
        for b in range(_B):
            soff = b * (_SEQ * _D) + off
            yield pltpu.make_async_copy(sbufs[k][b], out_hbm.at[pl.ds(soff, _C)], so[k])

    def start_in(t, k):
        for c in in_copies(t, k):
            c.start()

    def wait_in(t, k):
        for c in in_copies(t, k):
            c.wait()

    def start_out(t, k):
        for c in out_copies(t, k):
            c.start()

    def wait_out(t, k):
        for c in out_copies(t, k):
            c.wait()

    def compute(k):
        @plsc.parallel_loop(0, _C // _LANES, unroll=8)
        def _(i):
            s = pl.ds(i * _LANES, _LANES)
            v = tbufs[k][s]
            for b in range(_B):
                plsc.addupdate(sbufs[k][b].at[s], v)

    start_in(0, 0)
    start_in(1, 1)

    def ring_step(g, _):
        t0 = g * _NBUF
        for k in range(_NBUF):
            t = t0 + k

            @pl.when(t >= 2)
            def _():
                wait_out(t - 2, (k - 2) % _NBUF)

            @pl.when(t + 2 < _NTILES)
            def _():
                start_in(t + 2, (k + 2) % _NBUF)

            wait_in(t, k)
            compute(k)
            start_out(t, k)
        return 0

    lax.fori_loop(0, _NTILES // _NBUF, ring_step, 0)

    wait_out(_NTILES - 2, (_NTILES - 2) % _NBUF)
    wait_out(_NTILES - 1, (_NTILES - 1) % _NBUF)


def kernel(seq, table):
    seq_flat = seq.reshape(-1)
    table_flat = table.reshape(-1)
    scratch = (
        [pltpu.VMEM((_C,), jnp.float32) for _ in range(_NBUF)]
        + [pltpu.VMEM((_C,), jnp.float32) for _ in range(_NBUF * _B)]
        + [pltpu.SemaphoreType.DMA for _ in range(2 * _NBUF)]
    )
    k = functools.partial(
        pl.kernel,
        out_type=jax.ShapeDtypeStruct((_B * _SEQ * _D,), jnp.float32),
        mesh=plsc.VectorSubcoreMesh(
            core_axis_name="c", subcore_axis_name="s",
            num_cores=_NC, num_subcores=_NS),
        scratch_types=scratch,
    )(_sc_body)
    out_flat = k(seq_flat, table_flat)
    return out_flat.reshape(seq.shape)


# speedup vs baseline: 1.8335x; 1.8335x over previous
import functools

import jax
import jax.numpy as jnp
from jax import lax
from jax.experimental import pallas as pl
from jax.experimental.pallas import tpu as pltpu
from jax.experimental.pallas import tpu_sc as plsc

_NC = 2
_NS = 16
_NW = _NC * _NS
_LANES = 16

_B = 4
_SEQ = 8192
_D = 1024
_ROWS_PER_W = _SEQ // _NW
_T = 8
_NTILES = _ROWS_PER_W // _T
_NBUF = 6
_LEAD = 3
_MAIN = (_NTILES // _NBUF) * _NBUF

_TC_BLK = 1024


def _sc_body(seq_hbm, table_hbm, out_hbm, *scratch):
    tbufs = scratch[0:_NBUF]
    sbufs = scratch[_NBUF:2 * _NBUF]
    si = scratch[2 * _NBUF:3 * _NBUF]
    so = scratch[3 * _NBUF:]

    wid = lax.axis_index("s") * _NC + lax.axis_index("c")
    row_base = wid * _ROWS_PER_W

    def in_copies(t, k):
        r0 = row_base + t * _T
        yield pltpu.make_async_copy(
            table_hbm.at[pl.ds(r0, _T), :], tbufs[k], si[k])
        yield pltpu.make_async_copy(
            seq_hbm.at[0, pl.ds(r0, _T), :], sbufs[k], si[k])

    def out_copy(t, k):
        r0 = row_base + t * _T
        return pltpu.make_async_copy(
            sbufs[k], out_hbm.at[pl.ds(r0, _T), :], so[k])

    def start_in(t, k):
        for c in in_copies(t, k):
            c.start()

    def wait_in(t, k):
        for c in in_copies(t, k):
            c.wait()

    def compute(k):
        for r in range(_T):
            @plsc.parallel_loop(0, _D // _LANES, unroll=8)
            def _(i):
                s = pl.ds(i * _LANES, _LANES)
                plsc.addupdate(sbufs[k].at[r, s], tbufs[k][r, s])

    def tile_body(t, k, static):
        wait_in(t, k)
        compute(k)
        out_copy(t, k).start()
        if static:
            if t >= 2:
                out_copy(t - 2, (k - 2) % _NBUF).wait()
            if t + _LEAD < _NTILES:
                start_in(t + _LEAD, (k + _LEAD) % _NBUF)
        else:
            @pl.when(t >= 2)
            def _():
                out_copy(t - 2, (k - 2) % _NBUF).wait()

            @pl.when(t + _LEAD < _NTILES)
            def _():
                start_in(t + _LEAD, (k + _LEAD) % _NBUF)

    for t in range(_LEAD):
        start_in(t, t)

    def ring_step(g, _):
        t0 = g * _NBUF
        for k in range(_NBUF):
            tile_body(t0 + k, k, static=False)
        return 0

    lax.fori_loop(0, _MAIN // _NBUF, ring_step, 0)

    for t in range(_MAIN, _NTILES):
        tile_body(t, t % _NBUF, static=True)

    out_copy(_NTILES - 2, (_NTILES - 2) % _NBUF).wait()
    out_copy(_NTILES - 1, (_NTILES - 1) % _NBUF).wait()


def _tc_add(seq_ref, table_ref, out_ref):
    out_ref[...] = seq_ref[...] + table_ref[...]


def kernel(seq, table):
    sc_k = functools.partial(
        pl.kernel,
        out_type=jax.ShapeDtypeStruct((_SEQ, _D), jnp.float32),
        mesh=plsc.VectorSubcoreMesh(
            core_axis_name="c", subcore_axis_name="s",
            num_cores=_NC, num_subcores=_NS),
        scratch_types=(
            [pltpu.VMEM((_T, _D), jnp.float32) for _ in range(2 * _NBUF)]
            + [pltpu.SemaphoreType.DMA for _ in range(2 * _NBUF)]
        ),
        compiler_params=pltpu.CompilerParams(use_tc_tiling_on_sc=True),
    )(_sc_body)
    out_b0 = sc_k(seq, table)

    out_rest = pl.pallas_call(
        _tc_add,
        grid=(_SEQ // _TC_BLK, _B - 1),
        in_specs=[
            pl.BlockSpec((1, _TC_BLK, _D), lambda r, i: (i + 1, r, 0)),
            pl.BlockSpec((_TC_BLK, _D), lambda r, i: (r, 0)),
        ],
        out_specs=pl.BlockSpec((1, _TC_BLK, _D), lambda r, i: (i, r, 0)),
        out_shape=jax.ShapeDtypeStruct((_B - 1, _SEQ, _D), seq.dtype),
    )(seq, table)

    return jnp.concatenate([out_b0[None], out_rest], axis=0)


# speedup vs baseline: 2.9122x; 1.5884x over previous
import functools

import jax
import jax.numpy as jnp
from jax import lax
from jax.experimental import pallas as pl
from jax.experimental.pallas import tpu as pltpu
from jax.experimental.pallas import tpu_sc as plsc

_NC = 2
_NS = 16
_NW = _NC * _NS
_LANES = 16

_B = 4
_SEQ = 8192
_D = 1024
_ROWS_PER_W = _SEQ // _NW
_T = 8
_NTILES = _ROWS_PER_W // _T
_NBUF = 3
_MAIN = (_NTILES // _NBUF) * _NBUF


def _sc_body(seq_hbm, table_hbm, out_hbm, *scratch):
    tbufs = scratch[0:_NBUF]
    sbufs = [scratch[_NBUF + i * _B:_NBUF + (i + 1) * _B] for i in range(_NBUF)]
    si = scratch[_NBUF + _NBUF * _B:_NBUF + _NBUF * _B + _NBUF]
    so = scratch[_NBUF + _NBUF * _B + _NBUF:]

    wid = lax.axis_index("s") * _NC + lax.axis_index("c")
    row_base = wid * _ROWS_PER_W

    def in_copies(t, k):
        r0 = row_base + t * _T
        yield pltpu.make_async_copy(
            table_hbm.at[pl.ds(r0, _T), :], tbufs[k], si[k])
        for b in range(_B):
            yield pltpu.make_async_copy(
                seq_hbm.at[b, pl.ds(r0, _T), :], sbufs[k][b], si[k])

    def out_copies(t, k):
        r0 = row_base + t * _T
        for b in range(_B):
            yield pltpu.make_async_copy(
                sbufs[k][b], out_hbm.at[b, pl.ds(r0, _T), :], so[k])

    def start_in(t, k):
        for c in in_copies(t, k):
            c.start()

    def wait_in(t, k):
        for c in in_copies(t, k):
            c.wait()

    def start_out(t, k):
        for c in out_copies(t, k):
            c.start()

    def wait_out(t, k):
        for c in out_copies(t, k):
            c.wait()

    def compute(k):
        for r in range(_T):
            @plsc.parallel_loop(0, _D // _LANES, unroll=8)
            def _(i):
                s = pl.ds(i * _LANES, _LANES)
                v = tbufs[k][r, s]
                for b in range(_B):
                    plsc.addupdate(sbufs[k][b].at[r, s], v)

    def tile_body(t, k, static):
        wait_in(t, k)
        if static:
            if t >= 1:
                wait_out(t - 1, (k - 1) % _NBUF)
            if t + 2 < _NTILES:
                start_in(t + 2, (k + 2) % _NBUF)
        else:
            @pl.when(t >= 1)
            def _():
                wait_out(t - 1, (k - 1) % _NBUF)

            start_in(t + 2, (k + 2) % _NBUF)
        compute(k)
        start_out(t, k)

    start_in(0, 0)
    start_in(1, 1)

    def ring_step(g, _):
        t0 = g * _NBUF
        for k in range(_NBUF):
            tile_body(t0 + k, k, static=False)
        return 0

    lax.fori_loop(0, _MAIN // _NBUF, ring_step, 0)

    tile_body(_MAIN + 0, 0, static=True)
    tile_body(_MAIN + 1, 1, static=True)

    wait_out(_NTILES - 1, (_NTILES - 1) % _NBUF)


def kernel(seq, table):
    scratch = (
        [pltpu.VMEM((_T, _D), jnp.float32) for _ in range(_NBUF)]
        + [pltpu.VMEM((_T, _D), jnp.float32) for _ in range(_NBUF * _B)]
        + [pltpu.SemaphoreType.DMA for _ in range(2 * _NBUF)]
    )
    k = functools.partial(
        pl.kernel,
        out_type=jax.ShapeDtypeStruct((_B, _SEQ, _D), jnp.float32),
        mesh=plsc.VectorSubcoreMesh(
            core_axis_name="c", subcore_axis_name="s",
            num_cores=_NC, num_subcores=_NS),
        scratch_types=scratch,
        compiler_params=pltpu.CompilerParams(
            use_tc_tiling_on_sc=True,
            skip_device_barrier=True,
            disable_bounds_checks=True),
    )(_sc_body)
    return k(seq, table)
